# baseline (speedup 1.0000x reference)
import jax
import jax.numpy as jnp
from jax.experimental import pallas as pl
from jax.experimental.pallas import tpu as pltpu

_ROWS = 1000000
_COLS = 64
_B = 8000
_K = _ROWS // _B
_NBUF = 4


def _relu_stream_body(x_ref, y_ref, cache_hbm, out_hbm,
                      in_buf, out_buf, in_sems, out_sems):
    def in_copy(k, slot):
        return pltpu.make_async_copy(
            cache_hbm.at[pl.ds(k * _B, _B), :], in_buf.at[slot],
            in_sems.at[slot])

    def out_copy(k, slot):
        return pltpu.make_async_copy(
            out_buf.at[slot], out_hbm.at[pl.ds(k * _B, _B), :],
            out_sems.at[slot])

    for s in range(_NBUF):
        in_copy(s, s).start()

    def step(k, carry):
        slot = jax.lax.rem(k, _NBUF)
        in_copy(k, slot).wait()

        @pl.when(k >= _NBUF)
        def _():
            out_copy(k - _NBUF, slot).wait()

        out_buf[slot] = jnp.maximum(in_buf[slot], 0.0)

        @pl.when(k == 0)
        def _():
            m = x_ref[...] * y_ref[...]
            out_buf[0, 0:1, :] = jnp.maximum(m, 0.0)

        out_copy(k, slot).start()

        @pl.when(k + _NBUF < _K)
        def _():
            in_copy(k + _NBUF, slot).start()

        return carry

    jax.lax.fori_loop(0, _K, step, 0)

    for s in range(_NBUF):
        k = _K - _NBUF + s
        out_copy(k, k % _NBUF).wait()


def kernel(x, y, cache):
    return pl.pallas_call(
        _relu_stream_body,
        in_specs=[
            pl.BlockSpec(memory_space=pltpu.VMEM),
            pl.BlockSpec(memory_space=pltpu.VMEM),
            pl.BlockSpec(memory_space=pl.ANY),
        ],
        out_specs=pl.BlockSpec(memory_space=pl.ANY),
        out_shape=jax.ShapeDtypeStruct((_ROWS, _COLS), jnp.float32),
        scratch_shapes=[
            pltpu.VMEM((_NBUF, _B, _COLS), jnp.float32),
            pltpu.VMEM((_NBUF, _B, _COLS), jnp.float32),
            pltpu.SemaphoreType.DMA((_NBUF,)),
            pltpu.SemaphoreType.DMA((_NBUF,)),
        ],
    )(x, y, cache)
